# baseline (speedup 1.0000x reference)
import functools

import jax
import jax.numpy as jnp
from jax import lax
from jax.experimental import pallas as pl
from jax.experimental.pallas import tpu as pltpu
from jax.experimental.pallas import tpu_sc as plsc

_L = 16
_NC = 2
_NS = 16
_NW = _NC * _NS
_BLK = 512


def _pick_chunk(ew, maxc=128):
  for c in range(maxc, 7, -8):
    if ew % c == 0:
      return c
  raise ValueError(f"no valid chunk for per-worker edge count {ew}")


def _make_deg(N_pad, E):
  ew = E // _NW
  sub = next(d for d in range(2048, 7, -8) if ew % d == 0)
  nsub = ew // sub
  nvec = sub // _L
  mesh = plsc.VectorSubcoreMesh(core_axis_name="c", subcore_axis_name="s")

  @functools.partial(
      pl.kernel,
      out_type=jax.ShapeDtypeStruct((_NW, 1, N_pad), jnp.float32),
      mesh=mesh,
      scratch_types=[
          pltpu.VMEM((sub,), jnp.int32),
          pltpu.VMEM((N_pad,), jnp.float32),
      ],
      compiler_params=pltpu.CompilerParams(needs_layout_passes=False),
  )
  def deg_kernel(dst_hbm, deg_hbm, idx_v, hist_v):
    c = lax.axis_index("c")
    s = lax.axis_index("s")
    u = c * _NS + s

    one16 = jnp.ones((_L,), jnp.float32)
    zero16 = jnp.zeros((_L,), jnp.float32)

    def fill_zero(i, _):
      hist_v[pl.ds(i * _L, _L)] = zero16
      return 0

    lax.fori_loop(0, N_pad // _L, fill_zero, 0)

    def subblk(b2, _):
      pltpu.sync_copy(dst_hbm.at[pl.ds(u * ew + b2 * sub, sub)], idx_v)

      def step(j, _):
        idx16 = idx_v[pl.ds(j * _L, _L)]
        plsc.addupdate_scatter(hist_v, [idx16], one16)
        return 0

      lax.fori_loop(0, nvec, step, 0)
      return 0

    lax.fori_loop(0, nsub, subblk, 0)

    pltpu.sync_copy(hist_v, deg_hbm.at[u, 0])

  return deg_kernel


def _pick_block(steps):
  for b in range(50, 0, -1):
    if steps % b == 0:
      return b
  return 1


def _make_agg(N_pad, D, E):
  ew = E // _NW
  chunk = _pick_chunk(ew)
  steps = ew // chunk
  bs = _pick_block(steps)
  nblk = steps // bs
  rows = N_pad // _NS
  nbuf = 3
  dist = 1
  mesh = plsc.VectorSubcoreMesh(core_axis_name="c", subcore_axis_name="s")

  @functools.partial(
      pl.kernel,
      out_type=jax.ShapeDtypeStruct((_NC, N_pad, D), jnp.float32),
      mesh=mesh,
      scratch_types=[
          pltpu.VMEM((bs, chunk), jnp.int32),
          pltpu.VMEM((bs, chunk), jnp.int32),
          [pltpu.VMEM((chunk, D), jnp.float32)] * nbuf,
          [pltpu.SemaphoreType.DMA] * nbuf,
          [pltpu.SemaphoreType.DMA] * nbuf,
          pltpu.VMEM_SHARED((N_pad, D), jnp.float32),
      ],
  )
  def agg_kernel(hp_hbm, zeros_hbm, src_hbm, dst_hbm, out_hbm,
                 src_v, dst_v, bufs, gsems, ssems, acc_sh):
    c = lax.axis_index("c")
    s = lax.axis_index("s")
    u = c * _NS + s
    r0 = s * rows

    @pl.when(c == 0)
    def _():
      pltpu.sync_copy(hp_hbm.at[pl.ds(r0, rows)], acc_sh.at[pl.ds(r0, rows)])

    @pl.when(c != 0)
    def _():
      pltpu.sync_copy(zeros_hbm.at[pl.ds(r0, rows)],
                      acc_sh.at[pl.ds(r0, rows)])

    plsc.subcore_barrier()

    def gather(j, k):
      pltpu.async_copy(hp_hbm.at[src_v.at[j]], bufs[k], gsems[k])

    def gather_wait(j, k):
      pltpu.make_async_copy(hp_hbm.at[src_v.at[j]], bufs[k], gsems[k]).wait()

    def scatter(j, k):
      pltpu.async_copy(bufs[k], acc_sh.at[dst_v.at[j]], ssems[k], add=True)

    def scatter_wait(j, k):
      pltpu.make_async_copy(bufs[k], acc_sh.at[dst_v.at[j]], ssems[k]).wait()

    def block(b, _):
      pltpu.sync_copy(src_hbm.at[u, b], src_v)
      pltpu.sync_copy(dst_hbm.at[u, b], dst_v)
      for j in range(min(dist, bs)):
        gather(j, j % nbuf)
      for j in range(bs):
        k = j % nbuf
        gather_wait(j, k)
        scatter(j, k)
        jn = j + dist
        if jn < bs:
          kn = jn % nbuf
          if jn >= nbuf:
            scatter_wait(jn - nbuf, kn)
          gather(jn, kn)
      for j in range(max(0, bs - nbuf), bs):
        scatter_wait(j, j % nbuf)
      return 0

    lax.fori_loop(0, nblk, block, 0)
    plsc.subcore_barrier()
    pltpu.sync_copy(acc_sh.at[pl.ds(r0, rows)],
                    out_hbm.at[c, pl.ds(r0, rows)])

  return agg_kernel


def _scale_body(x_ref, w_ref, d_ref, hp_ref, dis_ref):
  deg = jnp.sum(d_ref[:, 0, :], axis=0) + 1.0
  dis = lax.rsqrt(deg)
  dis_ref[...] = dis
  h = jnp.dot(x_ref[...], w_ref[...], preferred_element_type=jnp.float32)
  hp_ref[...] = h * dis[:, None]


def _finish_body(p0_ref, p1_ref, dis_ref, b_ref, out_ref):
  acc = (p0_ref[0] + p1_ref[0]) * dis_ref[...][:, None] + b_ref[...][None, :]
  m = jnp.max(acc, axis=1, keepdims=True)
  lse = jnp.log(jnp.sum(jnp.exp(acc - m), axis=1, keepdims=True)) + m
  out_ref[...] = acc - lse


def kernel(x, edge_index, W, b):
  N, D_in = x.shape
  D = W.shape[1]
  E = edge_index.shape[1]
  unit = 2560
  N_pad = ((N + unit - 1) // unit) * unit
  grid = N_pad // _BLK

  src = edge_index[0].astype(jnp.int32)
  dst = edge_index[1].astype(jnp.int32)
  x_pad = jnp.pad(x, ((0, N_pad - N), (0, 0)))
  zeros2d = jnp.zeros((N_pad, D), jnp.float32)

  deg_parts3 = _make_deg(N_pad, E)(dst)

  hp, dis = pl.pallas_call(
      _scale_body,
      grid=(grid,),
      in_specs=[
          pl.BlockSpec((_BLK, D_in), lambda i: (i, 0)),
          pl.BlockSpec((D_in, D), lambda i: (0, 0)),
          pl.BlockSpec((_NW, 1, _BLK), lambda i: (0, 0, i)),
      ],
      out_specs=[
          pl.BlockSpec((_BLK, D), lambda i: (i, 0)),
          pl.BlockSpec((_BLK,), lambda i: (i,)),
      ],
      out_shape=[
          jax.ShapeDtypeStruct((N_pad, D), jnp.float32),
          jax.ShapeDtypeStruct((N_pad,), jnp.float32),
      ],
  )(x_pad, W, deg_parts3)

  agg_chunk = _pick_chunk(E // _NW)
  agg_steps = E // _NW // agg_chunk
  agg_bs = _pick_block(agg_steps)
  src4d = src.reshape(_NW, agg_steps // agg_bs, agg_bs, agg_chunk)
  dst4d = dst.reshape(_NW, agg_steps // agg_bs, agg_bs, agg_chunk)
  parts = _make_agg(N_pad, D, E)(hp, zeros2d, src4d, dst4d)

  out = pl.pallas_call(
      _finish_body,
      grid=(grid,),
      in_specs=[
          pl.BlockSpec((1, _BLK, D), lambda i: (0, i, 0)),
          pl.BlockSpec((1, _BLK, D), lambda i: (1, i, 0)),
          pl.BlockSpec((_BLK,), lambda i: (i,)),
          pl.BlockSpec((D,), lambda i: (0,)),
      ],
      out_specs=pl.BlockSpec((_BLK, D), lambda i: (i, 0)),
      out_shape=jax.ShapeDtypeStruct((N_pad, D), jnp.float32),
  )(parts, parts, dis, b)

  return out[:N]


# speedup vs baseline: 1.2773x; 1.2773x over previous
import functools

import jax
import jax.numpy as jnp
from jax import lax
from jax.experimental import pallas as pl
from jax.experimental.pallas import tpu as pltpu
from jax.experimental.pallas import tpu_sc as plsc

_L = 16
_NC = 2
_NS = 16
_NW = _NC * _NS
_BLK = 512


def _pick_chunk(ew, maxc=128):
  for c in range(maxc, 7, -8):
    if ew % c == 0:
      return c
  raise ValueError(f"no valid chunk for per-worker edge count {ew}")


def _make_deg(N_pad, E):
  ew = E // _NW
  sub = next(d for d in range(2048, 7, -8) if ew % d == 0)
  nsub = ew // sub
  nvec = sub // _L
  mesh = plsc.VectorSubcoreMesh(core_axis_name="c", subcore_axis_name="s")

  @functools.partial(
      pl.kernel,
      out_type=jax.ShapeDtypeStruct((_NW, 1, N_pad), jnp.float32),
      mesh=mesh,
      scratch_types=[
          pltpu.VMEM((sub,), jnp.int32),
          pltpu.VMEM((N_pad,), jnp.float32),
      ],
      compiler_params=pltpu.CompilerParams(needs_layout_passes=False),
  )
  def deg_kernel(dst_hbm, deg_hbm, idx_v, hist_v):
    c = lax.axis_index("c")
    s = lax.axis_index("s")
    u = c * _NS + s

    one16 = jnp.ones((_L,), jnp.float32)
    zero16 = jnp.zeros((_L,), jnp.float32)

    def fill_zero(i, _):
      hist_v[pl.ds(i * _L, _L)] = zero16
      return 0

    lax.fori_loop(0, N_pad // _L, fill_zero, 0)

    def subblk(b2, _):
      pltpu.sync_copy(dst_hbm.at[pl.ds(u * ew + b2 * sub, sub)], idx_v)

      def step(j, _):
        idx16 = idx_v[pl.ds(j * _L, _L)]
        plsc.addupdate_scatter(hist_v, [idx16], one16)
        return 0

      lax.fori_loop(0, nvec, step, 0)
      return 0

    lax.fori_loop(0, nsub, subblk, 0)

    pltpu.sync_copy(hist_v, deg_hbm.at[u, 0])

  return deg_kernel


def _pick_block(steps):
  for b in range(50, 0, -1):
    if steps % b == 0:
      return b
  return 1


def _make_agg(N_pad, D, E):
  ew = E // _NW
  chunk = _pick_chunk(ew)
  steps = ew // chunk
  bs = _pick_block(steps)
  nblk = steps // bs
  rows = N_pad // _NS
  nbuf = 3
  dist = 2
  mesh = plsc.VectorSubcoreMesh(core_axis_name="c", subcore_axis_name="s")

  @functools.partial(
      pl.kernel,
      out_type=jax.ShapeDtypeStruct((_NC, N_pad, D), jnp.float32),
      mesh=mesh,
      scratch_types=[
          pltpu.VMEM((bs, chunk), jnp.int32),
          pltpu.VMEM((bs, chunk), jnp.int32),
          [pltpu.VMEM((chunk, D), jnp.float32)] * nbuf,
          [pltpu.SemaphoreType.DMA] * nbuf,
          [pltpu.SemaphoreType.DMA] * nbuf,
          pltpu.VMEM_SHARED((N_pad, D), jnp.float32),
      ],
  )
  def agg_kernel(hp_hbm, zeros_hbm, src_hbm, dst_hbm, out_hbm,
                 src_v, dst_v, bufs, gsems, ssems, acc_sh):
    c = lax.axis_index("c")
    s = lax.axis_index("s")
    u = c * _NS + s
    r0 = s * rows

    @pl.when(c == 0)
    def _():
      pltpu.sync_copy(hp_hbm.at[pl.ds(r0, rows)], acc_sh.at[pl.ds(r0, rows)])

    @pl.when(c != 0)
    def _():
      pltpu.sync_copy(zeros_hbm.at[pl.ds(r0, rows)],
                      acc_sh.at[pl.ds(r0, rows)])

    plsc.subcore_barrier()

    def gather(j, k):
      pltpu.async_copy(hp_hbm.at[src_v.at[j]], bufs[k], gsems[k])

    def gather_wait(j, k):
      pltpu.make_async_copy(hp_hbm.at[src_v.at[j]], bufs[k], gsems[k]).wait()

    def scatter(j, k):
      pltpu.async_copy(bufs[k], acc_sh.at[dst_v.at[j]], ssems[k], add=True)

    def scatter_wait(j, k):
      pltpu.make_async_copy(bufs[k], acc_sh.at[dst_v.at[j]], ssems[k]).wait()

    def block(b, _):
      pltpu.sync_copy(src_hbm.at[u, b], src_v)
      pltpu.sync_copy(dst_hbm.at[u, b], dst_v)
      for j in range(min(dist, bs)):
        gather(j, j % nbuf)
      for j in range(bs):
        k = j % nbuf
        gather_wait(j, k)
        scatter(j, k)
        jn = j + dist
        if jn < bs:
          kn = jn % nbuf
          if jn >= nbuf:
            scatter_wait(jn - nbuf, kn)
          gather(jn, kn)
      for j in range(max(0, bs - nbuf), bs):
        scatter_wait(j, j % nbuf)
      return 0

    lax.fori_loop(0, nblk, block, 0)
    plsc.subcore_barrier()
    pltpu.sync_copy(acc_sh.at[pl.ds(r0, rows)],
                    out_hbm.at[c, pl.ds(r0, rows)])

  return agg_kernel


def _scale_body(x_ref, w_ref, d_ref, hp_ref, dis_ref):
  deg = jnp.sum(d_ref[:, 0, :], axis=0) + 1.0
  dis = lax.rsqrt(deg)
  dis_ref[...] = dis
  h = jnp.dot(x_ref[...], w_ref[...], preferred_element_type=jnp.float32)
  hp_ref[...] = h * dis[:, None]


def _finish_body(p0_ref, p1_ref, dis_ref, b_ref, out_ref):
  acc = (p0_ref[0] + p1_ref[0]) * dis_ref[...][:, None] + b_ref[...][None, :]
  m = jnp.max(acc, axis=1, keepdims=True)
  lse = jnp.log(jnp.sum(jnp.exp(acc - m), axis=1, keepdims=True)) + m
  out_ref[...] = acc - lse


def kernel(x, edge_index, W, b):
  N, D_in = x.shape
  D = W.shape[1]
  E = edge_index.shape[1]
  unit = 2560
  N_pad = ((N + unit - 1) // unit) * unit
  grid = N_pad // _BLK

  src = edge_index[0].astype(jnp.int32)
  dst = edge_index[1].astype(jnp.int32)
  x_pad = jnp.pad(x, ((0, N_pad - N), (0, 0)))
  zeros2d = jnp.zeros((N_pad, D), jnp.float32)

  deg_parts3 = _make_deg(N_pad, E)(dst)

  hp, dis = pl.pallas_call(
      _scale_body,
      grid=(grid,),
      in_specs=[
          pl.BlockSpec((_BLK, D_in), lambda i: (i, 0)),
          pl.BlockSpec((D_in, D), lambda i: (0, 0)),
          pl.BlockSpec((_NW, 1, _BLK), lambda i: (0, 0, i)),
      ],
      out_specs=[
          pl.BlockSpec((_BLK, D), lambda i: (i, 0)),
          pl.BlockSpec((_BLK,), lambda i: (i,)),
      ],
      out_shape=[
          jax.ShapeDtypeStruct((N_pad, D), jnp.float32),
          jax.ShapeDtypeStruct((N_pad,), jnp.float32),
      ],
  )(x_pad, W, deg_parts3)

  agg_chunk = _pick_chunk(E // _NW)
  agg_steps = E // _NW // agg_chunk
  agg_bs = _pick_block(agg_steps)
  src4d = src.reshape(_NW, agg_steps // agg_bs, agg_bs, agg_chunk)
  dst4d = dst.reshape(_NW, agg_steps // agg_bs, agg_bs, agg_chunk)
  parts = _make_agg(N_pad, D, E)(hp, zeros2d, src4d, dst4d)

  out = pl.pallas_call(
      _finish_body,
      grid=(grid,),
      in_specs=[
          pl.BlockSpec((1, _BLK, D), lambda i: (0, i, 0)),
          pl.BlockSpec((1, _BLK, D), lambda i: (1, i, 0)),
          pl.BlockSpec((_BLK,), lambda i: (i,)),
          pl.BlockSpec((D,), lambda i: (0,)),
      ],
      out_specs=pl.BlockSpec((_BLK, D), lambda i: (i, 0)),
      out_shape=jax.ShapeDtypeStruct((N_pad, D), jnp.float32),
  )(parts, parts, dis, b)

  return out[:N]
